# baseline (speedup 1.0000x reference)
import jax
import jax.numpy as jnp
from jax import lax
from jax.experimental import pallas as pl
from jax.experimental.pallas import tpu as pltpu
from jax.experimental.pallas import tpu_sc as plsc

FEATURE = 1000000
FEATURE_PAD = 1 << 20
BATCH = 16384
N_FIELDS = 26
NUM_CORES = 2
NUM_SUBCORES = 16
NUM_WORKERS = NUM_CORES * NUM_SUBCORES
ROWS_PER_W = BATCH // NUM_WORKERS
IDX_PER_W = ROWS_PER_W * N_FIELDS
LANES = 16


def _sc_body(w_hbm, idx_hbm, out_hbm, idx_v, rows_v, out_v, sem):
    wid = lax.axis_index("s") * NUM_CORES + lax.axis_index("c")
    base_o = wid * ROWS_PER_W

    for g in range(4):
        n = 4096 if g < 3 else 1024
        pltpu.sync_copy(
            idx_hbm.at[pl.ds((g * NUM_WORKERS + wid) * 4096, n)],
            idx_v.at[pl.ds(g * 4096, n)],
        )
    pltpu.async_copy(w_hbm.at[idx_v], rows_v, sem).wait()

    @pl.loop(0, ROWS_PER_W // LANES)
    def _chunk(i):
        b = i * LANES
        acc = rows_v[pl.ds(b, LANES)]
        for f in range(1, N_FIELDS):
            acc = acc + rows_v[pl.ds(f * ROWS_PER_W + b, LANES)]
        out_v[pl.ds(b, LANES)] = acc

    pltpu.sync_copy(out_v, out_hbm.at[pl.ds(base_o, ROWS_PER_W)])


@jax.jit
def kernel(inputs, w):
    idx_flat = (
        jnp.pad(inputs.astype(jnp.int32).T, ((0, 6), (0, 0)))
        .reshape(4, 8, NUM_WORKERS, ROWS_PER_W)
        .transpose(0, 2, 1, 3)
        .reshape(-1)
    )
    w_flat = jnp.pad(w, ((0, FEATURE_PAD - FEATURE), (0, 0))).reshape(-1)
    mesh = plsc.VectorSubcoreMesh(core_axis_name="c", subcore_axis_name="s")
    out = pl.kernel(
        _sc_body,
        out_type=jax.ShapeDtypeStruct((BATCH,), jnp.float32),
        mesh=mesh,
        scratch_types=[
            pltpu.VMEM((IDX_PER_W,), jnp.int32),
            pltpu.VMEM((IDX_PER_W,), jnp.float32),
            pltpu.VMEM((ROWS_PER_W,), jnp.float32),
            pltpu.SemaphoreType.DMA,
        ],
    )(w_flat, idx_flat)
    return out.reshape(BATCH, 1)


# speedup vs baseline: 1.1094x; 1.1094x over previous
import jax
import jax.numpy as jnp
from jax import lax
from jax.experimental import pallas as pl
from jax.experimental.pallas import tpu as pltpu
from jax.experimental.pallas import tpu_sc as plsc

FEATURE = 1000000
FEATURE_PAD = 1000448
BATCH = 16384
N_FIELDS = 26
NUM_CORES = 2
NUM_SUBCORES = 16
NUM_WORKERS = NUM_CORES * NUM_SUBCORES
ROWS_PER_W = BATCH // NUM_WORKERS
IDX_PER_W = ROWS_PER_W * N_FIELDS
LANES = 16
BG = 128
BG_PER_W = ROWS_PER_W // BG
BG_BLOCK = 32 * BG
BG_VALID = N_FIELDS * BG


def _sc_body(w_hbm, idx_hbm, out_hbm, idx_v, rows_v, out_v, s0, s1, s2, s3, gsem):
    wid = lax.axis_index("s") * NUM_CORES + lax.axis_index("c")
    base_o = wid * ROWS_PER_W

    sems = (s0, s1, s2, s3)
    copies = [
        pltpu.async_copy(
            idx_hbm.at[pl.ds((g * NUM_WORKERS + wid) * 4096, 4096 if g < 3 else 1024)],
            idx_v.at[pl.ds(g * 4096, 4096 if g < 3 else 1024)],
            sems[g],
        )
        for g in range(4)
    ]
    for c in copies:
        c.wait()
    pltpu.async_copy(w_hbm.at[idx_v], rows_v, gsem).wait()

    @pl.loop(0, ROWS_PER_W // LANES)
    def _chunk(i):
        b = i * LANES
        acc = rows_v[pl.ds(b, LANES)]
        for f in range(1, N_FIELDS):
            acc = acc + rows_v[pl.ds(f * ROWS_PER_W + b, LANES)]
        out_v[pl.ds(b, LANES)] = acc

    pltpu.sync_copy(out_v, out_hbm.at[pl.ds(base_o, ROWS_PER_W)])


@jax.jit
def kernel(inputs, w):
    idx_flat = (
        jnp.pad(inputs.astype(jnp.int32).T, ((0, 6), (0, 0)))
        .reshape(4, 8, NUM_WORKERS, ROWS_PER_W)
        .transpose(0, 2, 1, 3)
        .reshape(-1)
    )
    w_flat = jnp.pad(w, ((0, FEATURE_PAD - FEATURE), (0, 0))).reshape(-1)
    mesh = plsc.VectorSubcoreMesh(core_axis_name="c", subcore_axis_name="s")
    out = pl.kernel(
        _sc_body,
        out_type=jax.ShapeDtypeStruct((BATCH,), jnp.float32),
        mesh=mesh,
        scratch_types=[
            pltpu.VMEM((IDX_PER_W,), jnp.int32),
            pltpu.VMEM((IDX_PER_W,), jnp.float32),
            pltpu.VMEM((ROWS_PER_W,), jnp.float32),
            pltpu.SemaphoreType.DMA,
            pltpu.SemaphoreType.DMA,
            pltpu.SemaphoreType.DMA,
            pltpu.SemaphoreType.DMA,
            pltpu.SemaphoreType.DMA,
        ],
    )(w_flat, idx_flat)
    return out.reshape(BATCH, 1)
